# Initial kernel scaffold; baseline (speedup 1.0000x reference)
import functools

import jax
import jax.numpy as jnp
from jax import lax
from jax.experimental import pallas as pl
from jax.experimental.pallas import tpu as pltpu
from jax.experimental.pallas import tpu_sc as plsc

N = 10000
E = 320000
D = 128
NC = 2
NS = 16
NW = NC * NS
EPT = E // NW
G = 80
NCH = EPT // G
RPS = N // NS
NB = 1000


def _mm_body(x_ref, w_ref, a_ref, h_ref, al_ref):
    xb = x_ref[...]
    hs = []
    for k in range(4):
        hk = jnp.dot(xb, w_ref[k], preferred_element_type=jnp.float32)
        h_ref[k] = hk
        hs.append(hk)
    al = jnp.dot(hs[0], a_ref[0], preferred_element_type=jnp.float32)
    for k in range(1, 4):
        al = al + jnp.dot(hs[k], a_ref[k], preferred_element_type=jnp.float32)
    al_ref[...] = al


def _matmuls(x, wstack, apad):
    return pl.pallas_call(
        _mm_body,
        grid=(N // NB,),
        in_specs=[
            pl.BlockSpec((NB, D), lambda i: (i, 0)),
            pl.BlockSpec((4, D, D), lambda i: (0, 0, 0)),
            pl.BlockSpec((4, D, D), lambda i: (0, 0, 0)),
        ],
        out_specs=[
            pl.BlockSpec((4, NB, D), lambda i: (0, i, 0)),
            pl.BlockSpec((NB, D), lambda i: (i, 0)),
        ],
        out_shape=[
            jax.ShapeDtypeStruct((4, N, D), jnp.float32),
            jax.ShapeDtypeStruct((N, D), jnp.float32),
        ],
    )(x, wstack, apad)


def _make_sc_gat():
    mesh = plsc.VectorSubcoreMesh(core_axis_name="c", subcore_axis_name="s")

    @functools.partial(
        pl.kernel,
        mesh=mesh,
        out_type=[
            jax.ShapeDtypeStruct((NC, N, D), jnp.float32),
            jax.ShapeDtypeStruct((NW, N), jnp.float32),
        ],
        scratch_types=[
            pltpu.VMEM((N,), jnp.float32),
            pltpu.VMEM((N,), jnp.float32),
            pltpu.VMEM((NCH, G), jnp.int32),
            pltpu.VMEM((NCH, G), jnp.int32),
            pltpu.VMEM((N,), jnp.float32),
            pltpu.VMEM((G,), jnp.float32),
            pltpu.VMEM((G, D), jnp.float32),
            pltpu.VMEM_SHARED((N, D), jnp.float32),
            pltpu.SemaphoreType.DMA,
        ],
    )
    def gat(src_hbm, dst_hbm, als_hbm, ald_hbm, h_hbm, acc_out, s_out,
            als_v, ald_v, src_v, dst_v, spart_v, ex_v, rows_v, acc_sh, sem):
        cid = lax.axis_index("c")
        sid = lax.axis_index("s")
        wid = sid * NC + cid

        pltpu.sync_copy(als_hbm, als_v)
        pltpu.sync_copy(ald_hbm, ald_v)
        pltpu.sync_copy(src_hbm.at[wid], src_v)
        pltpu.sync_copy(dst_hbm.at[wid], dst_v)

        zero16 = jnp.zeros((16,), jnp.float32)

        def zs(i, carry):
            spart_v[pl.ds(i * 16, 16)] = zero16
            return carry
        lax.fori_loop(0, N // 16, zs, 0)

        def zr(i, carry):
            for c8 in range(8):
                rows_v[i, pl.ds(c8 * 16, 16)] = zero16
            return carry
        lax.fori_loop(0, G, zr, 0)

        rbase = sid * RPS
        for rep in range(RPS // G):
            pltpu.sync_copy(rows_v, acc_sh.at[pl.ds(rbase + rep * G, G)])
        rem = RPS - (RPS // G) * G
        pltpu.sync_copy(rows_v.at[pl.ds(0, rem)],
                        acc_sh.at[pl.ds(rbase + (RPS // G) * G, rem)])
        plsc.subcore_barrier()

        def chunk(g, carry):
            for q in range(G // 16):
                sv = src_v[g, pl.ds(q * 16, 16)]
                dv = dst_v[g, pl.ds(q * 16, 16)]
                z = plsc.load_gather(als_v, [sv]) + plsc.load_gather(ald_v, [dv])
                e = jnp.where(z >= 0.0, z, z * 0.2)
                ex = jnp.exp(e)
                ex_v[pl.ds(q * 16, 16)] = ex
                plsc.addupdate_scatter(spart_v, [dv], ex)
            pltpu.async_copy(h_hbm.at[src_v.at[g]], rows_v, sem).wait()

            def srow(r, c2):
                scale = plsc.load_gather(ex_v, [jnp.zeros((16,), jnp.int32) + r])
                for c8 in range(8):
                    rows_v[r, pl.ds(c8 * 16, 16)] = rows_v[r, pl.ds(c8 * 16, 16)] * scale
                return c2
            lax.fori_loop(0, G, srow, 0)
            pltpu.sync_copy(rows_v, acc_sh.at[dst_v.at[g]], add=True)
            return carry
        lax.fori_loop(0, NCH, chunk, 0)

        pltpu.sync_copy(spart_v, s_out.at[wid])
        plsc.subcore_barrier()
        pltpu.sync_copy(acc_sh.at[pl.ds(rbase, RPS)],
                        acc_out.at[cid, pl.ds(rbase, RPS)])

    return gat


_SC_GAT = _make_sc_gat()


def _combine_body(k, acc_ref, s_ref, al_ref, h_ref, b_ref, o_ref):
    als = al_ref[:, 2 * k:2 * k + 1]
    ald = al_ref[:, 2 * k + 1:2 * k + 2]
    z = als + ald
    e = jnp.where(z >= 0.0, z, z * 0.2)
    exl = jnp.exp(e)
    stot = jnp.sum(s_ref[...], axis=1, keepdims=True) + exl + 1e-16
    h = h_ref[...]
    num = acc_ref[0] + acc_ref[1] + exl * h
    o_ref[...] = jnp.maximum(num / stot + b_ref[...], 0.0)


def _combine(k, acc2, s_t, al, hk, bvec):
    return pl.pallas_call(
        functools.partial(_combine_body, k),
        grid=(N // NB,),
        in_specs=[
            pl.BlockSpec((NC, NB, D), lambda i: (0, i, 0)),
            pl.BlockSpec((NB, NW), lambda i: (i, 0)),
            pl.BlockSpec((NB, D), lambda i: (i, 0)),
            pl.BlockSpec((NB, D), lambda i: (i, 0)),
            pl.BlockSpec((1, D), lambda i: (0, 0)),
        ],
        out_specs=pl.BlockSpec((NB, D), lambda i: (i, 0)),
        out_shape=jax.ShapeDtypeStruct((N, D), jnp.float32),
    )(acc2, s_t, al, hk, bvec)


def kernel(x, edge_index, W_tp, as_tp, ad_tp, b_tp, W_tn, as_tn, ad_tn, b_tn,
           W_dp, as_dp, ad_dp, b_dp, W_dn, as_dn, ad_dn, b_dn):
    wstack = jnp.stack([W_tp, W_tn, W_dp, W_dn])
    a_sd = ((as_tp, ad_tp), (as_tn, ad_tn), (as_dp, ad_dp), (as_dn, ad_dn))
    apad = jnp.zeros((4, D, D), jnp.float32)
    for k, (a_s, a_d) in enumerate(a_sd):
        apad = apad.at[k, :, 2 * k].set(a_s).at[k, :, 2 * k + 1].set(a_d)
    bstack = (b_tp, b_tn, b_dp, b_dn)

    h4, al = _matmuls(x, wstack, apad)
    alt = al[:, :8].T

    kmap = (0, 1, 0, 1, 2, 3, 2, 3)
    outs = []
    for t in range(8):
        k = kmap[t]
        src3 = edge_index[t, 0].reshape(NW, NCH, G)
        dst3 = edge_index[t, 1].reshape(NW, NCH, G)
        acc2, spart = _SC_GAT(src3, dst3, alt[2 * k], alt[2 * k + 1], h4[k])
        outs.append(_combine(k, acc2, spart.T, al, h4[k],
                             bstack[k].reshape(1, D)))
    return (outs[0], outs[2], outs[4], outs[6], outs[1], outs[3], outs[5], outs[7])


# speedup vs baseline: 27.3251x; 27.3251x over previous
import functools

import jax
import jax.numpy as jnp
from jax import lax
from jax.experimental import pallas as pl
from jax.experimental.pallas import tpu as pltpu
from jax.experimental.pallas import tpu_sc as plsc

N = 10000
E = 320000
D = 128
NC = 2
NS = 16
NW = NC * NS
EPT = E // NW
G = 80
NCH = EPT // G
SB = 25
NSB = NCH // SB
RPS = N // NS
NB = 1000
KMAP = (0, 1, 0, 1, 2, 3, 2, 3)

_SC_PARAMS = pltpu.CompilerParams(
    needs_layout_passes=False, use_tc_tiling_on_sc=False)


def _mm_body(x_ref, w_ref, a_ref, h_ref, al_ref):
    xb = x_ref[...]
    hs = []
    for k in range(4):
        hk = jnp.dot(xb, w_ref[k], preferred_element_type=jnp.float32)
        h_ref[k] = hk
        hs.append(hk)
    al = jnp.dot(hs[0], a_ref[0], preferred_element_type=jnp.float32)
    for k in range(1, 4):
        al = al + jnp.dot(hs[k], a_ref[k], preferred_element_type=jnp.float32)
    al_ref[...] = al


def _matmuls(x, wstack, apad):
    return pl.pallas_call(
        _mm_body,
        grid=(N // NB,),
        in_specs=[
            pl.BlockSpec((NB, D), lambda i: (i, 0)),
            pl.BlockSpec((4, D, D), lambda i: (0, 0, 0)),
            pl.BlockSpec((4, D, D), lambda i: (0, 0, 0)),
        ],
        out_specs=[
            pl.BlockSpec((4, NB, D), lambda i: (0, i, 0)),
            pl.BlockSpec((NB, D), lambda i: (i, 0)),
        ],
        out_shape=[
            jax.ShapeDtypeStruct((4, N, D), jnp.float32),
            jax.ShapeDtypeStruct((N, D), jnp.float32),
        ],
    )(x, wstack, apad)


def _make_sc_scalar():
    mesh = plsc.VectorSubcoreMesh(core_axis_name="c", subcore_axis_name="s")

    @functools.partial(
        pl.kernel,
        mesh=mesh,
        compiler_params=_SC_PARAMS,
        out_type=[
            jax.ShapeDtypeStruct((8, NW, EPT), jnp.float32),
            jax.ShapeDtypeStruct((8, NW, N), jnp.float32),
        ],
        scratch_types=[
            pltpu.VMEM((N,), jnp.float32),
            pltpu.VMEM((N,), jnp.float32),
            pltpu.VMEM((NCH, G), jnp.int32),
            pltpu.VMEM((NCH, G), jnp.int32),
            pltpu.VMEM((N,), jnp.float32),
            pltpu.VMEM((EPT,), jnp.float32),
        ],
    )
    def scalar_phase(alt_hbm, src_hbm, dst_hbm, ex_out, s_out,
                     als_v, ald_v, src_v, dst_v, spart_v, ex_v):
        cid = lax.axis_index("c")
        sid = lax.axis_index("s")
        wid = sid * NC + cid
        zero16 = jnp.zeros((16,), jnp.float32)

        for t in range(8):
            k = KMAP[t]
            pltpu.sync_copy(alt_hbm.at[2 * k], als_v)
            pltpu.sync_copy(alt_hbm.at[2 * k + 1], ald_v)
            pltpu.sync_copy(src_hbm.at[t, wid], src_v)
            pltpu.sync_copy(dst_hbm.at[t, wid], dst_v)

            def zs(i, carry):
                spart_v[pl.ds(i * 16, 16)] = zero16
                return carry
            lax.fori_loop(0, N // 16, zs, 0)

            def chunk(g, carry):
                for q in range(G // 16):
                    sv = src_v[g, pl.ds(q * 16, 16)]
                    dv = dst_v[g, pl.ds(q * 16, 16)]
                    z = (plsc.load_gather(als_v, [sv])
                         + plsc.load_gather(ald_v, [dv]))
                    e = jnp.where(z >= 0.0, z, z * 0.2)
                    ex = jnp.exp(e)
                    ex_v[pl.ds(g * G + q * 16, 16)] = ex
                    plsc.addupdate_scatter(spart_v, [dv], ex)
                return carry
            lax.fori_loop(0, NCH, chunk, 0)

            pltpu.sync_copy(ex_v, ex_out.at[t, wid])
            pltpu.sync_copy(spart_v, s_out.at[t, wid])

    return scalar_phase


def _make_sc_rows():
    mesh = plsc.VectorSubcoreMesh(core_axis_name="c", subcore_axis_name="s")

    @functools.partial(
        pl.kernel,
        mesh=mesh,
        compiler_params=_SC_PARAMS,
        out_type=jax.ShapeDtypeStruct((8, NC, N, D), jnp.float32),
        scratch_types=[
            pltpu.VMEM((SB, G), jnp.int32),
            pltpu.VMEM((SB, G), jnp.int32),
            pltpu.VMEM((SB, G), jnp.float32),
            pltpu.VMEM((G, D), jnp.float32),
            pltpu.VMEM_SHARED((N, D), jnp.float32),
            pltpu.SemaphoreType.DMA,
        ],
    )
    def row_phase(src_hbm, dst_hbm, ex_hbm, h0, h1, h2, h3, acc_out,
                  src_sb, dst_sb, ex_sb, rows_v, acc_sh, sem):
        cid = lax.axis_index("c")
        sid = lax.axis_index("s")
        wid = sid * NC + cid
        zero16 = jnp.zeros((16,), jnp.float32)
        rbase = sid * RPS
        hs = (h0, h1, h2, h3)

        for t in range(8):
            h_hbm = hs[KMAP[t]]

            def zr(i, carry):
                for c8 in range(8):
                    rows_v[i, pl.ds(c8 * 16, 16)] = zero16
                return carry
            lax.fori_loop(0, G, zr, 0)
            for rep in range(RPS // G):
                pltpu.sync_copy(rows_v, acc_sh.at[pl.ds(rbase + rep * G, G)])
            rem = RPS - (RPS // G) * G
            pltpu.sync_copy(rows_v.at[pl.ds(0, rem)],
                            acc_sh.at[pl.ds(rbase + (RPS // G) * G, rem)])
            plsc.subcore_barrier()

            def superchunk(ss, carry):
                pltpu.sync_copy(src_hbm.at[t, wid, pl.ds(ss * SB, SB)], src_sb)
                pltpu.sync_copy(dst_hbm.at[t, wid, pl.ds(ss * SB, SB)], dst_sb)
                pltpu.sync_copy(ex_hbm.at[t, wid, pl.ds(ss * SB, SB)], ex_sb)

                def chunk(g, c2):
                    pltpu.async_copy(h_hbm.at[src_sb.at[g]], rows_v, sem).wait()

                    def srow(r, c3):
                        scale = plsc.load_gather(
                            ex_sb, [jnp.zeros((16,), jnp.int32) + g,
                                    jnp.zeros((16,), jnp.int32) + r])
                        for c8 in range(8):
                            rows_v[r, pl.ds(c8 * 16, 16)] = (
                                rows_v[r, pl.ds(c8 * 16, 16)] * scale)
                        return c3
                    lax.fori_loop(0, G, srow, 0)
                    pltpu.sync_copy(rows_v, acc_sh.at[dst_sb.at[g]], add=True)
                    return c2
                lax.fori_loop(0, SB, chunk, 0)
                return carry
            lax.fori_loop(0, NSB, superchunk, 0)

            plsc.subcore_barrier()
            pltpu.sync_copy(acc_sh.at[pl.ds(rbase, RPS)],
                            acc_out.at[t, cid, pl.ds(rbase, RPS)])

    return row_phase


_SC_SCALAR = _make_sc_scalar()
_SC_ROWS = _make_sc_rows()


def _combine_body(k, acc_ref, s_ref, al_ref, h_ref, b_ref, o_ref):
    als = al_ref[:, 2 * k:2 * k + 1]
    ald = al_ref[:, 2 * k + 1:2 * k + 2]
    z = als + ald
    e = jnp.where(z >= 0.0, z, z * 0.2)
    exl = jnp.exp(e)
    stot = jnp.sum(s_ref[...], axis=1, keepdims=True) + exl + 1e-16
    h = h_ref[...]
    num = acc_ref[0] + acc_ref[1] + exl * h
    o_ref[...] = jnp.maximum(num / stot + b_ref[...], 0.0)


def _combine(k, acc2, s_t, al, hk, bvec):
    return pl.pallas_call(
        functools.partial(_combine_body, k),
        grid=(N // NB,),
        in_specs=[
            pl.BlockSpec((NC, NB, D), lambda i: (0, i, 0)),
            pl.BlockSpec((NB, NW), lambda i: (i, 0)),
            pl.BlockSpec((NB, D), lambda i: (i, 0)),
            pl.BlockSpec((NB, D), lambda i: (i, 0)),
            pl.BlockSpec((1, D), lambda i: (0, 0)),
        ],
        out_specs=pl.BlockSpec((NB, D), lambda i: (i, 0)),
        out_shape=jax.ShapeDtypeStruct((N, D), jnp.float32),
    )(acc2, s_t, al, hk, bvec)


def kernel(x, edge_index, W_tp, as_tp, ad_tp, b_tp, W_tn, as_tn, ad_tn, b_tn,
           W_dp, as_dp, ad_dp, b_dp, W_dn, as_dn, ad_dn, b_dn):
    wstack = jnp.stack([W_tp, W_tn, W_dp, W_dn])
    a_sd = ((as_tp, ad_tp), (as_tn, ad_tn), (as_dp, ad_dp), (as_dn, ad_dn))
    apad = jnp.zeros((4, D, D), jnp.float32)
    for k, (a_s, a_d) in enumerate(a_sd):
        apad = apad.at[k, :, 2 * k].set(a_s).at[k, :, 2 * k + 1].set(a_d)
    bstack = (b_tp, b_tn, b_dp, b_dn)

    h4, al = _matmuls(x, wstack, apad)
    alt = al[:, :8].T

    srcs = edge_index[:, 0].reshape(8, NW, NCH, G)
    dsts = edge_index[:, 1].reshape(8, NW, NCH, G)

    ex_all, s_all = _SC_SCALAR(alt, srcs, dsts)
    acc_all = _SC_ROWS(srcs, dsts, ex_all.reshape(8, NW, NCH, G),
                       h4[0], h4[1], h4[2], h4[3])

    outs = []
    for t in range(8):
        k = KMAP[t]
        outs.append(_combine(k, acc_all[t], s_all[t].T, al, h4[k],
                             bstack[k].reshape(1, D)))
    return (outs[0], outs[2], outs[4], outs[6], outs[1], outs[3], outs[5], outs[7])


# speedup vs baseline: 45.4332x; 1.6627x over previous
import functools

import jax
import jax.numpy as jnp
from jax import lax
from jax.experimental import pallas as pl
from jax.experimental.pallas import tpu as pltpu
from jax.experimental.pallas import tpu_sc as plsc

N = 10000
E = 320000
D = 128
NC = 2
NS = 16
NW = NC * NS
EPT = E // NW
G = 80
NCH = EPT // G
SB = 25
NSB = NCH // SB
RPS = N // NS
NB = 1000
KMAP = (0, 1, 0, 1, 2, 3, 2, 3)

_SC_PARAMS = pltpu.CompilerParams(
    needs_layout_passes=False, use_tc_tiling_on_sc=False)


def _mm_body(x_ref, w_ref, a_ref, h_ref, al_ref):
    xb = x_ref[...]
    hs = []
    for k in range(4):
        hk = jnp.dot(xb, w_ref[k], preferred_element_type=jnp.float32)
        h_ref[k] = hk
        hs.append(hk)
    al = jnp.dot(hs[0], a_ref[0], preferred_element_type=jnp.float32)
    for k in range(1, 4):
        al = al + jnp.dot(hs[k], a_ref[k], preferred_element_type=jnp.float32)
    al_ref[...] = al


def _matmuls(x, wstack, apad):
    return pl.pallas_call(
        _mm_body,
        grid=(N // NB,),
        in_specs=[
            pl.BlockSpec((NB, D), lambda i: (i, 0)),
            pl.BlockSpec((4, D, D), lambda i: (0, 0, 0)),
            pl.BlockSpec((4, D, D), lambda i: (0, 0, 0)),
        ],
        out_specs=[
            pl.BlockSpec((4, NB, D), lambda i: (0, i, 0)),
            pl.BlockSpec((NB, D), lambda i: (i, 0)),
        ],
        out_shape=[
            jax.ShapeDtypeStruct((4, N, D), jnp.float32),
            jax.ShapeDtypeStruct((N, D), jnp.float32),
        ],
    )(x, wstack, apad)


def _make_sc_scalar():
    mesh = plsc.VectorSubcoreMesh(core_axis_name="c", subcore_axis_name="s")

    @functools.partial(
        pl.kernel,
        mesh=mesh,
        compiler_params=_SC_PARAMS,
        out_type=[
            jax.ShapeDtypeStruct((8, NW, EPT), jnp.float32),
            jax.ShapeDtypeStruct((8, NW, N), jnp.float32),
        ],
        scratch_types=[
            pltpu.VMEM((N,), jnp.float32),
            pltpu.VMEM((N,), jnp.float32),
            pltpu.VMEM((NCH, G), jnp.int32),
            pltpu.VMEM((NCH, G), jnp.int32),
            pltpu.VMEM((N,), jnp.float32),
            pltpu.VMEM((EPT,), jnp.float32),
        ],
    )
    def scalar_phase(alt_hbm, src_hbm, dst_hbm, ex_out, s_out,
                     als_v, ald_v, src_v, dst_v, spart_v, ex_v):
        cid = lax.axis_index("c")
        sid = lax.axis_index("s")
        wid = sid * NC + cid
        zero16 = jnp.zeros((16,), jnp.float32)

        for t in range(8):
            k = KMAP[t]
            pltpu.sync_copy(alt_hbm.at[2 * k], als_v)
            pltpu.sync_copy(alt_hbm.at[2 * k + 1], ald_v)
            pltpu.sync_copy(src_hbm.at[t, wid], src_v)
            pltpu.sync_copy(dst_hbm.at[t, wid], dst_v)

            def zs(i, carry):
                spart_v[pl.ds(i * 16, 16)] = zero16
                return carry
            lax.fori_loop(0, N // 16, zs, 0)

            def chunk(g, carry):
                for q in range(G // 16):
                    sv = src_v[g, pl.ds(q * 16, 16)]
                    dv = dst_v[g, pl.ds(q * 16, 16)]
                    z = (plsc.load_gather(als_v, [sv])
                         + plsc.load_gather(ald_v, [dv]))
                    e = jnp.where(z >= 0.0, z, z * 0.2)
                    ex = jnp.exp(e)
                    ex_v[pl.ds(g * G + q * 16, 16)] = ex
                    plsc.addupdate_scatter(spart_v, [dv], ex)
                return carry
            lax.fori_loop(0, NCH, chunk, 0)

            pltpu.sync_copy(ex_v, ex_out.at[t, wid])
            pltpu.sync_copy(spart_v, s_out.at[t, wid])

    return scalar_phase


def _make_sc_rows():
    mesh = plsc.VectorSubcoreMesh(core_axis_name="c", subcore_axis_name="s")

    @functools.partial(
        pl.kernel,
        mesh=mesh,
        compiler_params=_SC_PARAMS,
        out_type=jax.ShapeDtypeStruct((8, NC, N, D), jnp.float32),
        scratch_types=[
            pltpu.VMEM((SB, G), jnp.int32),
            pltpu.VMEM((SB, G), jnp.int32),
            pltpu.VMEM((SB, G), jnp.float32),
            pltpu.VMEM((G, D), jnp.float32),
            pltpu.VMEM((G, D), jnp.float32),
            pltpu.VMEM_SHARED((N, D), jnp.float32),
            pltpu.SemaphoreType.DMA,
            pltpu.SemaphoreType.DMA,
        ],
    )
    def row_phase(src_hbm, dst_hbm, ex_hbm, h0, h1, h2, h3, acc_out,
                  src_sb, dst_sb, ex_sb, rows0, rows1, acc_sh, sem0, sem1):
        cid = lax.axis_index("c")
        sid = lax.axis_index("s")
        wid = sid * NC + cid
        zero16 = jnp.zeros((16,), jnp.float32)
        rbase = sid * RPS
        hs = (h0, h1, h2, h3)

        def scale_rows(rows_v, gg):
            @plsc.parallel_loop(0, G, unroll=4)
            def _(r):
                scale = plsc.load_gather(
                    ex_sb, [jnp.zeros((16,), jnp.int32) + gg,
                            jnp.zeros((16,), jnp.int32) + r])
                for c8 in range(8):
                    rows_v[r, pl.ds(c8 * 16, 16)] = (
                        rows_v[r, pl.ds(c8 * 16, 16)] * scale)

        for t in range(8):
            h_hbm = hs[KMAP[t]]

            def zr(i, carry):
                for c8 in range(8):
                    rows0[i, pl.ds(c8 * 16, 16)] = zero16
                return carry
            lax.fori_loop(0, G, zr, 0)
            for rep in range(RPS // G):
                pltpu.sync_copy(rows0, acc_sh.at[pl.ds(rbase + rep * G, G)])
            rem = RPS - (RPS // G) * G
            pltpu.sync_copy(rows0.at[pl.ds(0, rem)],
                            acc_sh.at[pl.ds(rbase + (RPS // G) * G, rem)])
            plsc.subcore_barrier()

            def superchunk(ss, carry):
                pltpu.sync_copy(src_hbm.at[t, wid, pl.ds(ss * SB, SB)], src_sb)
                pltpu.sync_copy(dst_hbm.at[t, wid, pl.ds(ss * SB, SB)], dst_sb)
                pltpu.sync_copy(ex_hbm.at[t, wid, pl.ds(ss * SB, SB)], ex_sb)
                pltpu.async_copy(h_hbm.at[src_sb.at[0]], rows0, sem0)

                def pair(i, c2):
                    g0 = 2 * i
                    pltpu.make_async_copy(
                        h_hbm.at[src_sb.at[g0]], rows0, sem0).wait()
                    pltpu.async_copy(h_hbm.at[src_sb.at[g0 + 1]], rows1, sem1)
                    scale_rows(rows0, g0)
                    pltpu.sync_copy(rows0, acc_sh.at[dst_sb.at[g0]], add=True)
                    pltpu.make_async_copy(
                        h_hbm.at[src_sb.at[g0 + 1]], rows1, sem1).wait()
                    pltpu.async_copy(h_hbm.at[src_sb.at[g0 + 2]], rows0, sem0)
                    scale_rows(rows1, g0 + 1)
                    pltpu.sync_copy(rows1, acc_sh.at[dst_sb.at[g0 + 1]],
                                    add=True)
                    return c2
                lax.fori_loop(0, SB // 2, pair, 0)
                pltpu.make_async_copy(
                    h_hbm.at[src_sb.at[SB - 1]], rows0, sem0).wait()
                scale_rows(rows0, SB - 1)
                pltpu.sync_copy(rows0, acc_sh.at[dst_sb.at[SB - 1]], add=True)
                return carry
            lax.fori_loop(0, NSB, superchunk, 0)

            plsc.subcore_barrier()
            pltpu.sync_copy(acc_sh.at[pl.ds(rbase, RPS)],
                            acc_out.at[t, cid, pl.ds(rbase, RPS)])

    return row_phase


_SC_SCALAR = _make_sc_scalar()
_SC_ROWS = _make_sc_rows()


def _combine_body(k, acc_ref, s_ref, al_ref, h_ref, b_ref, o_ref):
    als = al_ref[:, 2 * k:2 * k + 1]
    ald = al_ref[:, 2 * k + 1:2 * k + 2]
    z = als + ald
    e = jnp.where(z >= 0.0, z, z * 0.2)
    exl = jnp.exp(e)
    stot = jnp.sum(s_ref[...], axis=1, keepdims=True) + exl + 1e-16
    h = h_ref[...]
    num = acc_ref[0] + acc_ref[1] + exl * h
    o_ref[...] = jnp.maximum(num / stot + b_ref[...], 0.0)


def _combine(k, acc2, s_t, al, hk, bvec):
    return pl.pallas_call(
        functools.partial(_combine_body, k),
        grid=(N // NB,),
        in_specs=[
            pl.BlockSpec((NC, NB, D), lambda i: (0, i, 0)),
            pl.BlockSpec((NB, NW), lambda i: (i, 0)),
            pl.BlockSpec((NB, D), lambda i: (i, 0)),
            pl.BlockSpec((NB, D), lambda i: (i, 0)),
            pl.BlockSpec((1, D), lambda i: (0, 0)),
        ],
        out_specs=pl.BlockSpec((NB, D), lambda i: (i, 0)),
        out_shape=jax.ShapeDtypeStruct((N, D), jnp.float32),
    )(acc2, s_t, al, hk, bvec)


def kernel(x, edge_index, W_tp, as_tp, ad_tp, b_tp, W_tn, as_tn, ad_tn, b_tn,
           W_dp, as_dp, ad_dp, b_dp, W_dn, as_dn, ad_dn, b_dn):
    wstack = jnp.stack([W_tp, W_tn, W_dp, W_dn])
    a_sd = ((as_tp, ad_tp), (as_tn, ad_tn), (as_dp, ad_dp), (as_dn, ad_dn))
    apad = jnp.zeros((4, D, D), jnp.float32)
    for k, (a_s, a_d) in enumerate(a_sd):
        apad = apad.at[k, :, 2 * k].set(a_s).at[k, :, 2 * k + 1].set(a_d)
    bstack = (b_tp, b_tn, b_dp, b_dn)

    h4, al = _matmuls(x, wstack, apad)
    alt = al[:, :8].T

    srcs = edge_index[:, 0].reshape(8, NW, NCH, G)
    dsts = edge_index[:, 1].reshape(8, NW, NCH, G)

    ex_all, s_all = _SC_SCALAR(alt, srcs, dsts)
    acc_all = _SC_ROWS(srcs, dsts, ex_all.reshape(8, NW, NCH, G),
                       h4[0], h4[1], h4[2], h4[3])

    outs = []
    for t in range(8):
        k = KMAP[t]
        outs.append(_combine(k, acc_all[t], s_all[t].T, al, h4[k],
                             bstack[k].reshape(1, D)))
    return (outs[0], outs[2], outs[4], outs[6], outs[1], outs[3], outs[5], outs[7])
